# baseline (speedup 1.0000x reference)
import functools

import jax
import jax.numpy as jnp
from jax import lax
from jax.experimental import pallas as pl
from jax.experimental.pallas import tpu as pltpu
from jax.experimental.pallas import tpu_sc as plsc

_GN = 128
_FEAT = 648
_FPAD = 768
_NSL = _FPAD // 32
_DW = 32
_LBASE = 3 * _GN * _GN
_NROW = _LBASE + 3 * _GN
_AABB_MIN = -2.5
_AABB_MAX = 2.5
_INV_EXT = 1.0 / (_AABB_MAX - _AABB_MIN + 1e-08)
_ACT_SHIFT = -1.0
_DENSITY_SCALE = 10.0

_NC = 2
_NS = 16
_NW = _NC * _NS
_CHUNK = 8


def _prep_body(x_ref, idx_ref, wts_ref):
    xr = x_ref[...]
    g = []
    for a in range(3):
        x01 = (xr[:, a] - _AABB_MIN) * _INV_EXT
        g.append(x01 * 2.0 - 1.0)
    plane_uv = ((g[0], g[1]), (g[0], g[2]), (g[1], g[2]))
    line_u = (g[2], g[1], g[0])
    for p in range(3):
        u, v = plane_uv[p]
        px = (u + 1.0) * (0.5 * (_GN - 1))
        py = (v + 1.0) * (0.5 * (_GN - 1))
        x0 = jnp.floor(px)
        y0 = jnp.floor(py)
        wx1 = px - x0
        wy1 = py - y0
        for k, (dy, dx) in enumerate(((0, 0), (0, 1), (1, 0), (1, 1))):
            yi = y0 + dy
            xi = x0 + dx
            inb = ((xi >= 0) & (xi <= _GN - 1) & (yi >= 0) & (yi <= _GN - 1))
            xc = jnp.clip(xi, 0, _GN - 1).astype(jnp.int32)
            yc = jnp.clip(yi, 0, _GN - 1).astype(jnp.int32)
            wk = (wy1 if dy else 1.0 - wy1) * (wx1 if dx else 1.0 - wx1)
            idx_ref[:, p * 4 + k] = p * (_GN * _GN) + yc * _GN + xc
            wts_ref[:, p * 4 + k] = wk * inb.astype(jnp.float32)
        pos = (line_u[p] + 1.0) * (0.5 * (_GN - 1))
        i0 = jnp.clip(jnp.floor(pos), 0, _GN - 2).astype(jnp.int32)
        w1 = pos - i0.astype(jnp.float32)
        idx_ref[:, 12 + 2 * p] = _LBASE + p * _GN + i0
        idx_ref[:, 12 + 2 * p + 1] = _LBASE + p * _GN + i0 + 1
        wts_ref[:, 12 + 2 * p] = 1.0 - w1
        wts_ref[:, 12 + 2 * p + 1] = w1


def _prep(x):
    b = x.shape[0]
    bt = 2048
    return pl.pallas_call(
        _prep_body,
        grid=(b // bt,),
        in_specs=[pl.BlockSpec((bt, 6), lambda i: (i, 0))],
        out_specs=(pl.BlockSpec((bt, 18), lambda i: (i, 0)),
                   pl.BlockSpec((bt, 18), lambda i: (i, 0))),
        out_shape=(jax.ShapeDtypeStruct((b, 18), jnp.int32),
                   jax.ShapeDtypeStruct((b, 18), jnp.float32)),
    )(x)


def _sc_body(a_hbm, d_hbm, iw_hbm, s_hbm,
             iw_v, ra, rd, sst,
             sem_a, sem_d, sem_iw, sem_o, pw):
    wid = lax.axis_index("s") * _NC + lax.axis_index("c")
    base = wid * pw
    nch = pw // _CHUNK

    _REC = 2 * _CHUNK * 18

    def load_iw(c, buf):
        start = (base // _CHUNK + c) * _REC
        pltpu.async_copy(iw_hbm.at[pl.ds(start, _REC)],
                         iw_v.at[buf], sem_iw.at[buf])

    def wait_iw(buf):
        pltpu.make_async_copy(iw_hbm.at[pl.ds(0, _REC)],
                              iw_v.at[buf], sem_iw.at[buf]).wait()

    def start_gather(buf):
        idx_ref = iw_v.at[buf, pl.ds(0, _CHUNK * 18)]
        pltpu.async_copy(a_hbm.at[idx_ref], ra.at[buf], sem_a.at[buf])
        pltpu.async_copy(d_hbm.at[idx_ref], rd.at[buf], sem_d.at[buf])

    def wait_gather(buf):
        idx_ref = iw_v.at[buf, pl.ds(0, _CHUNK * 18)]
        pltpu.make_async_copy(a_hbm.at[idx_ref], ra.at[buf],
                              sem_a.at[buf]).wait()
        pltpu.make_async_copy(d_hbm.at[idx_ref], rd.at[buf],
                              sem_d.at[buf]).wait()

    def compute_chunk(buf):
        rab = ra.at[buf]
        rdb = rd.at[buf]

        def point_body(i):
            o = i * 18
            wbase = _CHUNK * 18 + o
            w_lo = plsc.bitcast(iw_v[buf, pl.ds(wbase, 16)], jnp.float32)
            w_hi = plsc.bitcast(iw_v[buf, pl.ds(wbase + 2, 16)], jnp.float32)

            def bcast(kk):
                srcv, pos = (w_lo, kk) if kk < 16 else (w_hi, kk - 2)
                w16 = lax.gather(
                    srcv, jnp.full((16, 1), pos, jnp.int32),
                    lax.GatherDimensionNumbers(
                        offset_dims=(), collapsed_slice_dims=(0,),
                        start_index_map=(0,)),
                    (1,), mode=lax.GatherScatterMode.PROMISE_IN_BOUNDS)
                return plsc.pack(w16, w16, format=plsc.PackFormat.INTERLEAVED)

            wv = [bcast(kk) for kk in range(18)]
            feat = None
            for j in range(_NSL):
                sl = pl.ds(j * 32, 32)
                s = None
                for p in range(3):
                    acc = wv[4 * p] * rab[o + 4 * p, sl]
                    for kk in range(1, 4):
                        acc = acc + wv[4 * p + kk] * rab[o + 4 * p + kk, sl]
                    lac = (wv[12 + 2 * p] * rab[o + 12 + 2 * p, sl]
                           + wv[13 + 2 * p] * rab[o + 13 + 2 * p, sl])
                    term = acc * lac
                    s = term if s is None else s + term
                feat = s if feat is None else feat + s
            sst[buf, i, pl.ds(0, _DW)] = feat
            dsum = None
            for p in range(3):
                accd = wv[4 * p] * rdb[o + 4 * p, :]
                for kk in range(1, 4):
                    accd = accd + wv[4 * p + kk] * rdb[o + 4 * p + kk, :]
                lacd = (wv[12 + 2 * p] * rdb[o + 12 + 2 * p, :]
                        + wv[13 + 2 * p] * rdb[o + 13 + 2 * p, :])
                term = accd * lacd
                dsum = term if dsum is None else dsum + term
            sst[buf, i, pl.ds(_DW, _DW)] = dsum

        plsc.parallel_loop(0, _CHUNK)(point_body)

    def start_out(c, buf):
        start = base + c * _CHUNK
        pltpu.async_copy(sst.at[buf], s_hbm.at[pl.ds(start, _CHUNK)],
                         sem_o.at[buf])

    def wait_out(buf):
        pltpu.make_async_copy(sst.at[buf], s_hbm.at[pl.ds(0, _CHUNK)],
                              sem_o.at[buf]).wait()

    def iter_step(c, buf):
        nbuf = 1 - buf
        wait_iw(nbuf)
        start_gather(nbuf)
        wait_gather(buf)

        @pl.when(c >= 2)
        def _():
            wait_out(buf)

        compute_chunk(buf)
        start_out(c, buf)
        load_iw(jnp.minimum(c + 2, nch - 1), buf)

    load_iw(0, 0)
    wait_iw(0)
    start_gather(0)
    load_iw(1, 1)

    def pair_body(c2, carry):
        iter_step(2 * c2, 0)
        iter_step(2 * c2 + 1, 1)
        return carry

    lax.fori_loop(0, nch // 2, pair_body, 0)
    wait_out(0)
    wait_out(1)
    wait_iw(1)
    wait_gather(0)


def _sc_gather_combine(table_a, table_d, iw_flat):
    b = iw_flat.shape[0] // 36
    pw = b // _NW
    mesh = plsc.VectorSubcoreMesh(core_axis_name="c", subcore_axis_name="s")
    kfn = functools.partial(
        pl.kernel,
        out_type=jax.ShapeDtypeStruct((b, 2 * _DW), jnp.bfloat16),
        mesh=mesh,
        scratch_types=(
            pltpu.VMEM((2, 2 * _CHUNK * 18), jnp.int32),
            pltpu.VMEM((2, _CHUNK * 18, _FPAD), jnp.bfloat16),
            pltpu.VMEM((2, _CHUNK * 18, _DW), jnp.bfloat16),
            pltpu.VMEM((2, _CHUNK, 2 * _DW), jnp.bfloat16),
            pltpu.SemaphoreType.DMA((2,)),
            pltpu.SemaphoreType.DMA((2,)),
            pltpu.SemaphoreType.DMA((2,)),
            pltpu.SemaphoreType.DMA((2,)),
        ),
        compiler_params=pltpu.CompilerParams(
            use_tc_tiling_on_sc=False, needs_layout_passes=False),
    )(functools.partial(_sc_body, pw=pw))
    return kfn(table_a, table_d, iw_flat)


def _finish_body(x_ref, s_ref, w1e_ref, w1d_ref, b1_ref, w2_ref,
                 b2_ref, out_ref):
    xr = x_ref[...]
    sv = s_ref[...][:, : _DW].astype(jnp.float32)
    ddv = s_ref[...][:, _DW:].astype(jnp.float32)
    dx, dy, dz = xr[:, 3], xr[:, 4], xr[:, 5]
    nrm = jnp.maximum(jnp.sqrt(dx * dx + dy * dy + dz * dz), 1e-12)
    w1d = w1d_ref[...]
    hdir = ((dx / nrm)[:, None] * w1d[0][None, :]
            + (dy / nrm)[:, None] * w1d[1][None, :]
            + (dz / nrm)[:, None] * w1d[2][None, :])
    h = jnp.dot(sv, w1e_ref[...], preferred_element_type=jnp.float32)
    h = jnp.maximum(h + hdir + b1_ref[...][None, :], 0.0)
    t = jnp.dot(h, w2_ref[...], preferred_element_type=jnp.float32)
    t = t + b2_ref[...][None, :]
    rgb = 1.0 / (1.0 + jnp.exp(-t))
    valid = None
    for a in range(3):
        x01 = (xr[:, a] - _AABB_MIN) * _INV_EXT
        va = (x01 >= 0.0) & (x01 <= 1.0)
        valid = va if valid is None else (valid & va)
    z = jnp.sum(ddv, axis=1) + _ACT_SHIFT
    sp = jnp.maximum(z, 0.0) + jnp.log(1.0 + jnp.exp(-jnp.abs(z)))
    sigma = sp * _DENSITY_SCALE * valid.astype(jnp.float32)
    out_ref[...] = jnp.concatenate([rgb, sigma[:, None]], axis=1)


def _finish(x, s, w1e, w1d, b1, w2, b2):
    b = x.shape[0]
    bt = 2048
    grid = b // bt
    return pl.pallas_call(
        _finish_body,
        grid=(grid,),
        in_specs=[
            pl.BlockSpec((bt, 6), lambda i: (i, 0)),
            pl.BlockSpec((bt, 2 * _DW), lambda i: (i, 0)),
            pl.BlockSpec((_DW, 64), lambda i: (0, 0)),
            pl.BlockSpec((3, 64), lambda i: (0, 0)),
            pl.BlockSpec((64,), lambda i: (0,)),
            pl.BlockSpec((64, 3), lambda i: (0, 0)),
            pl.BlockSpec((3,), lambda i: (0,)),
        ],
        out_specs=pl.BlockSpec((bt, 4), lambda i: (i, 0)),
        out_shape=jax.ShapeDtypeStruct((b, 4), jnp.float32),
    )(x, s, w1e, w1d, b1, w2, b2)


def kernel(x, dp0, dp1, dp2, dl0, dl1, dl2, ap0, ap1, ap2, al0, al1, al2,
           W1, b1, W2, b2):
    def plane_t(ap):
        c, r, h, w = ap.shape
        t = ap.transpose(2, 3, 1, 0)
        t = jnp.pad(t, ((0, 0), (0, 0), (0, 0), (0, _DW - c)))
        return t.reshape(h * w, r * _DW).astype(jnp.bfloat16)

    def line_t(al):
        c, r, n = al.shape
        t = al.transpose(2, 1, 0)
        t = jnp.pad(t, ((0, 0), (0, 0), (0, _DW - c)))
        return t.reshape(n, r * _DW).astype(jnp.bfloat16)

    def dpad(t):
        return jnp.pad(t, ((0, 0), (0, _DW - 16))).astype(jnp.bfloat16)

    table_a = jnp.concatenate(
        [plane_t(ap0), plane_t(ap1), plane_t(ap2),
         line_t(al2), line_t(al1), line_t(al0)], axis=0)
    table_d = jnp.concatenate(
        [dpad(dp0.transpose(1, 2, 0).reshape(-1, 16)),
         dpad(dp1.transpose(1, 2, 0).reshape(-1, 16)),
         dpad(dp2.transpose(1, 2, 0).reshape(-1, 16)),
         dpad(dl2.T), dpad(dl1.T), dpad(dl0.T)], axis=0)

    w1e = jnp.pad(W1[:27], ((0, _DW - 27), (0, 0)))
    w1d = W1[27:30]

    idx, wts = _prep(x)
    b = x.shape[0]
    rec = jnp.concatenate(
        [idx.reshape(b // _CHUNK, _CHUNK * 18),
         jax.lax.bitcast_convert_type(wts, jnp.int32)
            .reshape(b // _CHUNK, _CHUNK * 18)], axis=1)
    s = _sc_gather_combine(table_a, table_d, rec.reshape(-1))
    return _finish(x, s, w1e, w1d, b1, W2, b2)


# speedup vs baseline: 1.0042x; 1.0042x over previous
import functools

import jax
import jax.numpy as jnp
from jax import lax
from jax.experimental import pallas as pl
from jax.experimental.pallas import tpu as pltpu
from jax.experimental.pallas import tpu_sc as plsc

_GN = 128
_FEAT = 648
_FPAD = 768
_NSL = _FPAD // 32
_DW = 32
_LBASE = 3 * _GN * _GN
_NROW = _LBASE + 3 * _GN
_AABB_MIN = -2.5
_AABB_MAX = 2.5
_INV_EXT = 1.0 / (_AABB_MAX - _AABB_MIN + 1e-08)
_ACT_SHIFT = -1.0
_DENSITY_SCALE = 10.0

_NC = 2
_NS = 16
_NW = _NC * _NS
_CHUNK = 8


def _prep_body(x_ref, idx_ref, wts_ref):
    xr = x_ref[...]
    g = []
    for a in range(3):
        x01 = (xr[:, a] - _AABB_MIN) * _INV_EXT
        g.append(x01 * 2.0 - 1.0)
    plane_uv = ((g[0], g[1]), (g[0], g[2]), (g[1], g[2]))
    line_u = (g[2], g[1], g[0])
    for p in range(3):
        u, v = plane_uv[p]
        px = (u + 1.0) * (0.5 * (_GN - 1))
        py = (v + 1.0) * (0.5 * (_GN - 1))
        x0 = jnp.floor(px)
        y0 = jnp.floor(py)
        wx1 = px - x0
        wy1 = py - y0
        for k, (dy, dx) in enumerate(((0, 0), (0, 1), (1, 0), (1, 1))):
            yi = y0 + dy
            xi = x0 + dx
            inb = ((xi >= 0) & (xi <= _GN - 1) & (yi >= 0) & (yi <= _GN - 1))
            xc = jnp.clip(xi, 0, _GN - 1).astype(jnp.int32)
            yc = jnp.clip(yi, 0, _GN - 1).astype(jnp.int32)
            wk = (wy1 if dy else 1.0 - wy1) * (wx1 if dx else 1.0 - wx1)
            idx_ref[:, p * 4 + k] = p * (_GN * _GN) + yc * _GN + xc
            wts_ref[:, p * 4 + k] = wk * inb.astype(jnp.float32)
        pos = (line_u[p] + 1.0) * (0.5 * (_GN - 1))
        i0 = jnp.clip(jnp.floor(pos), 0, _GN - 2).astype(jnp.int32)
        w1 = pos - i0.astype(jnp.float32)
        idx_ref[:, 12 + 2 * p] = _LBASE + p * _GN + i0
        idx_ref[:, 12 + 2 * p + 1] = _LBASE + p * _GN + i0 + 1
        wts_ref[:, 12 + 2 * p] = 1.0 - w1
        wts_ref[:, 12 + 2 * p + 1] = w1


def _prep(x):
    b = x.shape[0]
    bt = 2048
    return pl.pallas_call(
        _prep_body,
        grid=(b // bt,),
        in_specs=[pl.BlockSpec((bt, 6), lambda i: (i, 0))],
        out_specs=(pl.BlockSpec((bt, 18), lambda i: (i, 0)),
                   pl.BlockSpec((bt, 18), lambda i: (i, 0))),
        out_shape=(jax.ShapeDtypeStruct((b, 18), jnp.int32),
                   jax.ShapeDtypeStruct((b, 18), jnp.float32)),
    )(x)


def _sc_body(a_hbm, d_hbm, idx_hbm, wts_hbm, s_hbm,
             idx_v, wts_v, ra, rd, sst,
             sem_a, sem_d, sem_iw, sem_o, pw):
    wid = lax.axis_index("s") * _NC + lax.axis_index("c")
    base = wid * pw
    nch = pw // _CHUNK

    def load_iw(c, buf):
        start = (base + c * _CHUNK) * 18
        pltpu.async_copy(idx_hbm.at[pl.ds(start, _CHUNK * 18)],
                         idx_v.at[buf], sem_iw.at[buf])
        pltpu.async_copy(wts_hbm.at[pl.ds(start, _CHUNK * 18)],
                         wts_v.at[buf], sem_iw.at[buf])

    def wait_iw(buf):
        pltpu.make_async_copy(idx_hbm.at[pl.ds(0, _CHUNK * 18)],
                              idx_v.at[buf], sem_iw.at[buf]).wait()
        pltpu.make_async_copy(wts_hbm.at[pl.ds(0, _CHUNK * 18)],
                              wts_v.at[buf], sem_iw.at[buf]).wait()

    def start_gather(buf):
        pltpu.async_copy(a_hbm.at[idx_v.at[buf]], ra.at[buf], sem_a.at[buf])
        pltpu.async_copy(d_hbm.at[idx_v.at[buf]], rd.at[buf], sem_d.at[buf])

    def wait_gather(buf):
        pltpu.make_async_copy(a_hbm.at[idx_v.at[buf]], ra.at[buf],
                              sem_a.at[buf]).wait()
        pltpu.make_async_copy(d_hbm.at[idx_v.at[buf]], rd.at[buf],
                              sem_d.at[buf]).wait()

    def compute_chunk(buf):
        rab = ra.at[buf]
        rdb = rd.at[buf]

        def point_body(i):
            o = i * 18
            w_lo = wts_v[buf, pl.ds(o, 16)]
            w_hi = wts_v[buf, pl.ds(o + 2, 16)]

            def bcast(kk):
                srcv, pos = (w_lo, kk) if kk < 16 else (w_hi, kk - 2)
                w16 = lax.gather(
                    srcv, jnp.full((16, 1), pos, jnp.int32),
                    lax.GatherDimensionNumbers(
                        offset_dims=(), collapsed_slice_dims=(0,),
                        start_index_map=(0,)),
                    (1,), mode=lax.GatherScatterMode.PROMISE_IN_BOUNDS)
                return plsc.pack(w16, w16, format=plsc.PackFormat.INTERLEAVED)

            wv = [bcast(kk) for kk in range(18)]
            feat = None
            for j in range(_NSL):
                sl = pl.ds(j * 32, 32)
                s = None
                for p in range(3):
                    acc = wv[4 * p] * rab[o + 4 * p, sl]
                    for kk in range(1, 4):
                        acc = acc + wv[4 * p + kk] * rab[o + 4 * p + kk, sl]
                    lac = (wv[12 + 2 * p] * rab[o + 12 + 2 * p, sl]
                           + wv[13 + 2 * p] * rab[o + 13 + 2 * p, sl])
                    term = acc * lac
                    s = term if s is None else s + term
                feat = s if feat is None else feat + s
            sst[buf, i, pl.ds(0, _DW)] = feat
            dsum = None
            for p in range(3):
                accd = wv[4 * p] * rdb[o + 4 * p, :]
                for kk in range(1, 4):
                    accd = accd + wv[4 * p + kk] * rdb[o + 4 * p + kk, :]
                lacd = (wv[12 + 2 * p] * rdb[o + 12 + 2 * p, :]
                        + wv[13 + 2 * p] * rdb[o + 13 + 2 * p, :])
                term = accd * lacd
                dsum = term if dsum is None else dsum + term
            sst[buf, i, pl.ds(_DW, _DW)] = dsum

        plsc.parallel_loop(0, _CHUNK)(point_body)

    def start_out(c, buf):
        start = base + c * _CHUNK
        pltpu.async_copy(sst.at[buf], s_hbm.at[pl.ds(start, _CHUNK)],
                         sem_o.at[buf])

    def wait_out(buf):
        pltpu.make_async_copy(sst.at[buf], s_hbm.at[pl.ds(0, _CHUNK)],
                              sem_o.at[buf]).wait()

    def iter_step(c, buf):
        nbuf = 1 - buf
        wait_iw(nbuf)
        start_gather(nbuf)
        wait_gather(buf)

        @pl.when(c >= 2)
        def _():
            wait_out(buf)

        compute_chunk(buf)
        start_out(c, buf)
        load_iw(jnp.minimum(c + 2, nch - 1), buf)

    load_iw(0, 0)
    wait_iw(0)
    start_gather(0)
    load_iw(1, 1)

    def pair_body(c2, carry):
        iter_step(2 * c2, 0)
        iter_step(2 * c2 + 1, 1)
        return carry

    lax.fori_loop(0, nch // 2, pair_body, 0)
    wait_out(0)
    wait_out(1)
    wait_iw(1)
    wait_gather(0)


def _sc_gather_combine(table_a, table_d, idx_flat, wts_flat):
    b = idx_flat.shape[0] // 18
    pw = b // _NW
    mesh = plsc.VectorSubcoreMesh(core_axis_name="c", subcore_axis_name="s")
    kfn = functools.partial(
        pl.kernel,
        out_type=jax.ShapeDtypeStruct((b, 2 * _DW), jnp.bfloat16),
        mesh=mesh,
        scratch_types=(
            pltpu.VMEM((2, _CHUNK * 18), jnp.int32),
            pltpu.VMEM((2, _CHUNK * 18), jnp.float32),
            pltpu.VMEM((2, _CHUNK * 18, _FPAD), jnp.bfloat16),
            pltpu.VMEM((2, _CHUNK * 18, _DW), jnp.bfloat16),
            pltpu.VMEM((2, _CHUNK, 2 * _DW), jnp.bfloat16),
            pltpu.SemaphoreType.DMA((2,)),
            pltpu.SemaphoreType.DMA((2,)),
            pltpu.SemaphoreType.DMA((2,)),
            pltpu.SemaphoreType.DMA((2,)),
        ),
        compiler_params=pltpu.CompilerParams(
            use_tc_tiling_on_sc=False, needs_layout_passes=False),
    )(functools.partial(_sc_body, pw=pw))
    return kfn(table_a, table_d, idx_flat, wts_flat)


def _finish_body(x_ref, s_ref, w1e_ref, w1d_ref, b1_ref, w2_ref,
                 b2_ref, out_ref):
    xr = x_ref[...]
    sv = s_ref[...][:, : _DW].astype(jnp.float32)
    ddv = s_ref[...][:, _DW:].astype(jnp.float32)
    dx, dy, dz = xr[:, 3], xr[:, 4], xr[:, 5]
    nrm = jnp.maximum(jnp.sqrt(dx * dx + dy * dy + dz * dz), 1e-12)
    w1d = w1d_ref[...]
    hdir = ((dx / nrm)[:, None] * w1d[0][None, :]
            + (dy / nrm)[:, None] * w1d[1][None, :]
            + (dz / nrm)[:, None] * w1d[2][None, :])
    h = jnp.dot(sv, w1e_ref[...], preferred_element_type=jnp.float32)
    h = jnp.maximum(h + hdir + b1_ref[...][None, :], 0.0)
    t = jnp.dot(h, w2_ref[...], preferred_element_type=jnp.float32)
    t = t + b2_ref[...][None, :]
    rgb = 1.0 / (1.0 + jnp.exp(-t))
    valid = None
    for a in range(3):
        x01 = (xr[:, a] - _AABB_MIN) * _INV_EXT
        va = (x01 >= 0.0) & (x01 <= 1.0)
        valid = va if valid is None else (valid & va)
    z = jnp.sum(ddv, axis=1) + _ACT_SHIFT
    sp = jnp.maximum(z, 0.0) + jnp.log(1.0 + jnp.exp(-jnp.abs(z)))
    sigma = sp * _DENSITY_SCALE * valid.astype(jnp.float32)
    out_ref[...] = jnp.concatenate([rgb, sigma[:, None]], axis=1)


def _finish(x, s, w1e, w1d, b1, w2, b2):
    b = x.shape[0]
    bt = 1024
    grid = b // bt
    return pl.pallas_call(
        _finish_body,
        grid=(grid,),
        in_specs=[
            pl.BlockSpec((bt, 6), lambda i: (i, 0)),
            pl.BlockSpec((bt, 2 * _DW), lambda i: (i, 0)),
            pl.BlockSpec((_DW, 64), lambda i: (0, 0)),
            pl.BlockSpec((3, 64), lambda i: (0, 0)),
            pl.BlockSpec((64,), lambda i: (0,)),
            pl.BlockSpec((64, 3), lambda i: (0, 0)),
            pl.BlockSpec((3,), lambda i: (0,)),
        ],
        out_specs=pl.BlockSpec((bt, 4), lambda i: (i, 0)),
        out_shape=jax.ShapeDtypeStruct((b, 4), jnp.float32),
    )(x, s, w1e, w1d, b1, w2, b2)


def kernel(x, dp0, dp1, dp2, dl0, dl1, dl2, ap0, ap1, ap2, al0, al1, al2,
           W1, b1, W2, b2):
    def plane_t(ap):
        c, r, h, w = ap.shape
        t = ap.transpose(2, 3, 1, 0)
        t = jnp.pad(t, ((0, 0), (0, 0), (0, 0), (0, _DW - c)))
        return t.reshape(h * w, r * _DW).astype(jnp.bfloat16)

    def line_t(al):
        c, r, n = al.shape
        t = al.transpose(2, 1, 0)
        t = jnp.pad(t, ((0, 0), (0, 0), (0, _DW - c)))
        return t.reshape(n, r * _DW).astype(jnp.bfloat16)

    def dpad(t):
        return jnp.pad(t, ((0, 0), (0, _DW - 16))).astype(jnp.bfloat16)

    table_a = jnp.concatenate(
        [plane_t(ap0), plane_t(ap1), plane_t(ap2),
         line_t(al2), line_t(al1), line_t(al0)], axis=0)
    table_d = jnp.concatenate(
        [dpad(dp0.transpose(1, 2, 0).reshape(-1, 16)),
         dpad(dp1.transpose(1, 2, 0).reshape(-1, 16)),
         dpad(dp2.transpose(1, 2, 0).reshape(-1, 16)),
         dpad(dl2.T), dpad(dl1.T), dpad(dl0.T)], axis=0)

    w1e = jnp.pad(W1[:27], ((0, _DW - 27), (0, 0)))
    w1d = W1[27:30]

    idx, wts = _prep(x)
    s = _sc_gather_combine(table_a, table_d, idx.reshape(-1),
                           wts.reshape(-1))
    return _finish(x, s, w1e, w1d, b1, W2, b2)
